# baseline (speedup 1.0000x reference)
import functools

import jax
import jax.numpy as jnp
from jax import lax
from jax.experimental import pallas as pl
from jax.experimental.pallas import tpu as pltpu
from jax.experimental.pallas import tpu_sc as plsc

_B = 32
_N = 16384
_L = 16
_CHUNKS = _N // _L

_NEG_INF = float("-inf")
_BIG_I32 = 2**31 - 1


def _lane_iota():
    return lax.iota(jnp.int32, _L)


def _argmax_combine(bv, bi, lanes):
    for s in (8, 4, 2, 1):
        perm = lanes ^ s
        ov = bv.at[perm].get(mode="promise_in_bounds")
        oi = bi.at[perm].get(mode="promise_in_bounds")
        take = (ov > bv) | ((ov == bv) & (oi < bi))
        bv = jnp.where(take, ov, bv)
        bi = jnp.where(take, oi, bi)
    return bi


def _fps_body(xyz_hbm, out_hbm, x_v, y_v, z_v, stage_v):
    c = lax.axis_index("c")
    s = lax.axis_index("s")
    b = s * 2 + c

    pltpu.sync_copy(xyz_hbm.at[b, 0], x_v)
    pltpu.sync_copy(xyz_hbm.at[b, 1], y_v)
    pltpu.sync_copy(xyz_hbm.at[b, 2], z_v)

    lanes = _lane_iota()

    def step1(i, carry):
        bv, bi = carry
        v = y_v[pl.ds(i * _L, _L)]
        idx = lanes + i * _L
        take = v > bv
        return jnp.where(take, v, bv), jnp.where(take, idx, bi)

    bv, bi = lax.fori_loop(
        0, _CHUNKS, step1,
        (jnp.full((_L,), _NEG_INF, jnp.float32), jnp.zeros((_L,), jnp.int32)),
    )
    idxv = _argmax_combine(bv, bi, lanes)

    cx = plsc.load_gather(x_v, [idxv])
    cy = plsc.load_gather(y_v, [idxv])
    cz = plsc.load_gather(z_v, [idxv])

    def step2(i, carry):
        bv, bi = carry
        sl = pl.ds(i * _L, _L)
        dx = x_v[sl] - cx
        dy = y_v[sl] - cy
        dz = z_v[sl] - cz
        d = dx * dx + dy * dy + dz * dz
        d = jnp.minimum(d, jnp.float32(1e10))
        idx = lanes + i * _L
        take = d > bv
        return jnp.where(take, d, bv), jnp.where(take, idx, bi)

    bv2, bi2 = lax.fori_loop(
        0, _CHUNKS, step2,
        (jnp.full((_L,), _NEG_INF, jnp.float32), jnp.zeros((_L,), jnp.int32)),
    )
    idxv2 = _argmax_combine(bv2, bi2, lanes)

    stage_v[...] = jnp.where(lanes == 0, idxv, idxv2)
    pltpu.sync_copy(stage_v, out_hbm.at[b])


@jax.jit
def kernel(xyz):
    pts = xyz.reshape(_B, 3, _N)
    mesh = plsc.VectorSubcoreMesh(core_axis_name="c", subcore_axis_name="s")
    fps = pl.kernel(
        _fps_body,
        out_type=jax.ShapeDtypeStruct((_B, _L), jnp.int32),
        mesh=mesh,
        compiler_params=pltpu.CompilerParams(use_tc_tiling_on_sc=False, needs_layout_passes=False),
        scratch_types=[
            pltpu.VMEM((_N,), jnp.float32),
            pltpu.VMEM((_N,), jnp.float32),
            pltpu.VMEM((_N,), jnp.float32),
            pltpu.VMEM((_L,), jnp.int32),
        ],
    )
    out = fps(pts)
    return out[:, :2]


# speedup vs baseline: 1.0868x; 1.0868x over previous
import jax
import jax.numpy as jnp
from jax import lax
from jax.experimental import pallas as pl
from jax.experimental.pallas import tpu as pltpu
from jax.experimental.pallas import tpu_sc as plsc

_B = 32
_N = 16384
_L = 16
_U = 8
_STEP = _U * _L

_NEG_INF = float("-inf")


def _combine(a, b):
    av, ai = a
    bv, bi = b
    take = (bv > av) | ((bv == av) & (bi < ai))
    return jnp.where(take, bv, av), jnp.where(take, bi, ai)


def _argmax_lanes(bv, bi, lanes):
    for s in (8, 4, 2, 1):
        perm = lanes ^ s
        ov = bv.at[perm].get(mode="promise_in_bounds")
        oi = bi.at[perm].get(mode="promise_in_bounds")
        bv, bi = _combine((bv, bi), (ov, oi))
    return bi


def _sweep_argmax(accs, lanes):
    while len(accs) > 1:
        accs = [_combine(accs[k], accs[k + 1]) for k in range(0, len(accs), 2)]
    bv, bi = accs[0]
    return _argmax_lanes(bv, bi, lanes)


def _init_accs():
    return tuple(
        (jnp.full((_L,), _NEG_INF, jnp.float32), jnp.zeros((_L,), jnp.int32))
        for _ in range(_U)
    )


def _fps_body(xyz_hbm, out_hbm, x_v, y_v, z_v, stage_v, sem_x, sem_y, sem_z):
    c = lax.axis_index("c")
    s = lax.axis_index("s")
    b = s * 2 + c

    cp_y = pltpu.async_copy(xyz_hbm.at[b, 1], y_v, sem_y)
    cp_x = pltpu.async_copy(xyz_hbm.at[b, 0], x_v, sem_x)
    cp_z = pltpu.async_copy(xyz_hbm.at[b, 2], z_v, sem_z)
    cp_y.wait()

    lanes = lax.iota(jnp.int32, _L)

    @plsc.parallel_loop(0, _N, step=_STEP, carry=_init_accs())
    def pass1(i, accs):
        out = []
        for k in range(_U):
            v = y_v[pl.ds(i + k * _L, _L)]
            idx = lanes + (i + k * _L)
            out.append(_combine(accs[k], (v, idx)))
        return tuple(out)

    idxv = _sweep_argmax(list(pass1), lanes)

    cx = plsc.load_gather(x_v, [idxv])
    cy = plsc.load_gather(y_v, [idxv])
    cz = plsc.load_gather(z_v, [idxv])

    cp_x.wait()
    cp_z.wait()

    @plsc.parallel_loop(0, _N, step=_STEP, carry=_init_accs())
    def pass2(i, accs):
        out = []
        for k in range(_U):
            sl = pl.ds(i + k * _L, _L)
            dx = x_v[sl] - cx
            dy = y_v[sl] - cy
            dz = z_v[sl] - cz
            d = dx * dx + dy * dy + dz * dz
            d = jnp.minimum(d, jnp.float32(1e10))
            idx = lanes + (i + k * _L)
            out.append(_combine(accs[k], (d, idx)))
        return tuple(out)

    idxv2 = _sweep_argmax(list(pass2), lanes)

    stage_v[...] = jnp.where(lanes == 0, idxv, idxv2)
    pltpu.sync_copy(stage_v, out_hbm.at[b])


@jax.jit
def kernel(xyz):
    pts = xyz.reshape(_B, 3, _N)
    mesh = plsc.VectorSubcoreMesh(core_axis_name="c", subcore_axis_name="s")
    fps = pl.kernel(
        _fps_body,
        out_type=jax.ShapeDtypeStruct((_B, _L), jnp.int32),
        mesh=mesh,
        compiler_params=pltpu.CompilerParams(
            use_tc_tiling_on_sc=False, needs_layout_passes=False
        ),
        scratch_types=[
            pltpu.VMEM((_N,), jnp.float32),
            pltpu.VMEM((_N,), jnp.float32),
            pltpu.VMEM((_N,), jnp.float32),
            pltpu.VMEM((_L,), jnp.int32),
            pltpu.SemaphoreType.DMA,
            pltpu.SemaphoreType.DMA,
            pltpu.SemaphoreType.DMA,
        ],
    )
    out = fps(pts)
    return out[:, :2]


# speedup vs baseline: 1.0883x; 1.0014x over previous
import jax
import jax.numpy as jnp
from jax import lax
from jax.experimental import pallas as pl
from jax.experimental.pallas import tpu as pltpu
from jax.experimental.pallas import tpu_sc as plsc

_B = 32
_N = 16384
_L = 16
_U = 8
_STEP = _U * _L

_NEG_INF = float("-inf")


def _combine(a, b):
    av, ai = a
    bv, bi = b
    take = (bv > av) | ((bv == av) & (bi < ai))
    return jnp.where(take, bv, av), jnp.where(take, bi, ai)


def _argmax_lanes(bv, bi, lanes):
    for s in (8, 4, 2, 1):
        perm = lanes ^ s
        ov = bv.at[perm].get(mode="promise_in_bounds")
        oi = bi.at[perm].get(mode="promise_in_bounds")
        bv, bi = _combine((bv, bi), (ov, oi))
    return bi


def _sweep_argmax(accs, lanes):
    while len(accs) > 1:
        accs = [_combine(accs[k], accs[k + 1]) for k in range(0, len(accs), 2)]
    bv, bi = accs[0]
    return _argmax_lanes(bv, bi, lanes)


def _init_accs():
    return tuple(
        (jnp.full((_L,), _NEG_INF, jnp.float32), jnp.zeros((_L,), jnp.int32))
        for _ in range(_U)
    )


_R = _N // 128


def _fps_body(xyz_hbm, out_hbm, x_v, y_v, z_v, stage_v, sem_x, sem_y, sem_z):
    c = lax.axis_index("c")
    s = lax.axis_index("s")
    b = s * 2 + c

    base = b * (3 * _R)
    cp_y = pltpu.async_copy(xyz_hbm.at[pl.ds(base + _R, _R)], y_v, sem_y)
    cp_x = pltpu.async_copy(xyz_hbm.at[pl.ds(base, _R)], x_v, sem_x)
    cp_z = pltpu.async_copy(xyz_hbm.at[pl.ds(base + 2 * _R, _R)], z_v, sem_z)
    cp_y.wait()

    lanes = lax.iota(jnp.int32, _L)

    @plsc.parallel_loop(0, _R, carry=_init_accs())
    def pass1(q, accs):
        out = []
        for k in range(_U):
            v = y_v[q, pl.ds(k * _L, _L)]
            idx = lanes + (q * 128 + k * _L)
            out.append(_combine(accs[k], (v, idx)))
        return tuple(out)

    idxv = _sweep_argmax(list(pass1), lanes)

    iq = lax.shift_right_logical(idxv, 7)
    ic = idxv & 127
    cx = plsc.load_gather(x_v, [iq, ic])
    cy = plsc.load_gather(y_v, [iq, ic])
    cz = plsc.load_gather(z_v, [iq, ic])

    cp_x.wait()
    cp_z.wait()

    @plsc.parallel_loop(0, _R, carry=_init_accs())
    def pass2(q, accs):
        out = []
        for k in range(_U):
            sl = pl.ds(k * _L, _L)
            dx = x_v[q, sl] - cx
            dy = y_v[q, sl] - cy
            dz = z_v[q, sl] - cz
            d = dx * dx + dy * dy + dz * dz
            d = jnp.minimum(d, jnp.float32(1e10))
            idx = lanes + (q * 128 + k * _L)
            out.append(_combine(accs[k], (d, idx)))
        return tuple(out)

    idxv2 = _sweep_argmax(list(pass2), lanes)

    stage_v[...] = jnp.where(lanes == 0, idxv, idxv2)
    pltpu.sync_copy(stage_v, out_hbm.at[b])


@jax.jit
def kernel(xyz):
    pts = xyz.reshape(_B * 3 * (_N // 128), 128)
    mesh = plsc.VectorSubcoreMesh(core_axis_name="c", subcore_axis_name="s")
    fps = pl.kernel(
        _fps_body,
        out_type=jax.ShapeDtypeStruct((_B, _L), jnp.int32),
        mesh=mesh,
        compiler_params=pltpu.CompilerParams(
            use_tc_tiling_on_sc=False, needs_layout_passes=False
        ),
        scratch_types=[
            pltpu.VMEM((_N // 128, 128), jnp.float32),
            pltpu.VMEM((_N // 128, 128), jnp.float32),
            pltpu.VMEM((_N // 128, 128), jnp.float32),
            pltpu.VMEM((_L,), jnp.int32),
            pltpu.SemaphoreType.DMA,
            pltpu.SemaphoreType.DMA,
            pltpu.SemaphoreType.DMA,
        ],
    )
    out = fps(pts)
    return out[:, :2]


# speedup vs baseline: 1.2932x; 1.1883x over previous
import jax
import jax.numpy as jnp
from jax import lax
from jax.experimental import pallas as pl
from jax.experimental.pallas import tpu as pltpu
from jax.experimental.pallas import tpu_sc as plsc

_B = 32
_N = 16384
_L = 16
_U = 8
_C = 2048
_NCH = _N // _C

_NEG_INF = float("-inf")


def _sel(take, a, b):
    return jnp.where(take, a, b)


def _take_mask(a, b):
    av, ai = a[0], a[1]
    bv, bi = b[0], b[1]
    return (bv > av) | ((bv == av) & (bi < ai))


def _combine(a, b):
    take = _take_mask(a, b)
    return tuple(_sel(take, y, x) for x, y in zip(a, b))


def _butterfly(acc, lanes):
    for s in (8, 4, 2, 1):
        perm = lanes ^ s
        other = tuple(
            v.at[perm].get(mode="promise_in_bounds") for v in acc
        )
        acc = _combine(acc, other)
    return acc


def _tree_combine(accs):
    while len(accs) > 1:
        accs = [_combine(accs[k], accs[k + 1]) for k in range(0, len(accs), 2)]
    return accs[0]


def _scan_chunk_y(buf, base, accs, lanes):
    @plsc.parallel_loop(0, _C, step=_U * _L, carry=accs)
    def loop(i, accs):
        out = []
        for k in range(_U):
            sl = pl.ds(i + k * _L, _L)
            v = buf[1, sl]
            idx = lanes + (base + i + k * _L)
            cand = (v, idx, buf[0, sl], buf[2, sl])
            out.append(_combine(accs[k], cand))
        return tuple(out)

    return loop


def _scan_chunk_d(buf, base, accs, lanes, cx, cy, cz):
    @plsc.parallel_loop(0, _C, step=_U * _L, carry=accs)
    def loop(i, accs):
        out = []
        for k in range(_U):
            sl = pl.ds(i + k * _L, _L)
            dx = buf[0, sl] - cx
            dy = buf[1, sl] - cy
            dz = buf[2, sl] - cz
            d = dx * dx + dy * dy + dz * dz
            d = jnp.minimum(d, jnp.float32(1e10))
            idx = lanes + (base + i + k * _L)
            out.append(_combine(accs[k], (d, idx)))
        return tuple(out)

    return loop


def _fps_body(xyz_hbm, out_hbm, buf_a, buf_b, stage_v, sem_a, sem_b):
    c = lax.axis_index("c")
    s = lax.axis_index("s")
    b = s * 2 + c

    bufs = (buf_a, buf_b)
    sems = (sem_a, sem_b)
    lanes = lax.iota(jnp.int32, _L)

    def src(j):
        return xyz_hbm.at[0, b, :, pl.ds(j * _C, _C)]

    def stream_pass(init_accs, scan):
        cps = [None, None]
        cps[0] = pltpu.async_copy(src(0), bufs[0], sems[0])
        cps[1] = pltpu.async_copy(src(1), bufs[1], sems[1])
        accs = init_accs
        for j in range(_NCH):
            p = j % 2
            cps[p].wait()
            accs = scan(bufs[p], j * _C, accs)
            if j + 2 < _NCH:
                cps[p] = pltpu.async_copy(src(j + 2), bufs[p], sems[p])
        return accs

    def init1():
        return tuple(
            (
                jnp.full((_L,), _NEG_INF, jnp.float32),
                jnp.zeros((_L,), jnp.int32),
                jnp.zeros((_L,), jnp.float32),
                jnp.zeros((_L,), jnp.float32),
            )
            for _ in range(_U)
        )

    accs1 = stream_pass(init1(), lambda buf, base, a: _scan_chunk_y(buf, base, a, lanes))
    cy, idxv, cx, cz = _butterfly(_tree_combine(list(accs1)), lanes)

    def init2():
        return tuple(
            (
                jnp.full((_L,), _NEG_INF, jnp.float32),
                jnp.zeros((_L,), jnp.int32),
            )
            for _ in range(_U)
        )

    accs2 = stream_pass(
        init2(),
        lambda buf, base, a: _scan_chunk_d(buf, base, a, lanes, cx, cy, cz),
    )
    _, idxv2 = _butterfly(_tree_combine(list(accs2)), lanes)

    stage_v[...] = jnp.where(lanes == 0, idxv, idxv2)
    pltpu.sync_copy(stage_v, out_hbm.at[b])


@jax.jit
def kernel(xyz):
    mesh = plsc.VectorSubcoreMesh(core_axis_name="c", subcore_axis_name="s")
    fps = pl.kernel(
        _fps_body,
        out_type=jax.ShapeDtypeStruct((_B, _L), jnp.int32),
        mesh=mesh,
        compiler_params=pltpu.CompilerParams(
            use_tc_tiling_on_sc=True, needs_layout_passes=False
        ),
        scratch_types=[
            pltpu.VMEM((3, _C), jnp.float32),
            pltpu.VMEM((3, _C), jnp.float32),
            pltpu.VMEM((_L,), jnp.int32),
            pltpu.SemaphoreType.DMA,
            pltpu.SemaphoreType.DMA,
        ],
    )
    out = fps(xyz)
    return out[:, :2]


# speedup vs baseline: 1.4254x; 1.1022x over previous
import jax
import jax.numpy as jnp
from jax import lax
from jax.experimental import pallas as pl
from jax.experimental.pallas import tpu as pltpu
from jax.experimental.pallas import tpu_sc as plsc

_B = 32
_N = 16384
_L = 16
_U = 8
_C = 2048
_NCH = _N // _C
_NBUF = 7

_NEG_INF = float("-inf")


def _combine(a, b):
    take = (b[0] > a[0]) | ((b[0] == a[0]) & (b[1] < a[1]))
    return tuple(jnp.where(take, y, x) for x, y in zip(a, b))


def _butterfly(acc, lanes):
    for s in (8, 4, 2, 1):
        perm = lanes ^ s
        other = tuple(v.at[perm].get(mode="promise_in_bounds") for v in acc)
        acc = _combine(acc, other)
    return acc


def _tree_combine(accs):
    accs = list(accs)
    while len(accs) > 1:
        accs = [_combine(accs[k], accs[k + 1]) for k in range(0, len(accs), 2)]
    return accs[0]


def _scan_chunk_y(buf, base, accs, lanes):
    @plsc.parallel_loop(0, _C, step=_U * _L, carry=accs)
    def loop(i, accs):
        out = []
        for k in range(_U):
            sl = pl.ds(i + k * _L, _L)
            cand = (buf[1, sl], lanes + (base + i + k * _L), buf[0, sl], buf[2, sl])
            out.append(_combine(accs[k], cand))
        return tuple(out)

    return loop


def _scan_chunk_d(buf, base, accs, lanes, cx, cy, cz):
    @plsc.parallel_loop(0, _C, step=_U * _L, carry=accs)
    def loop(i, accs):
        out = []
        for k in range(_U):
            sl = pl.ds(i + k * _L, _L)
            dx = buf[0, sl] - cx
            dy = buf[1, sl] - cy
            dz = buf[2, sl] - cz
            d = dx * dx + dy * dy + dz * dz
            d = jnp.minimum(d, jnp.float32(1e10))
            out.append(_combine(accs[k], (d, lanes + (base + i + k * _L))))
        return tuple(out)

    return loop


def _fps_body(xyz_hbm, out_hbm, b0, b1, b2, b3, b4, b5, b6, stage_v,
              s0, s1, s2, s3, s4, s5, s6):
    c = lax.axis_index("c")
    s = lax.axis_index("s")
    b = s * 2 + c

    bufs = (b0, b1, b2, b3, b4, b5, b6)
    sems = (s0, s1, s2, s3, s4, s5, s6)
    lanes = lax.iota(jnp.int32, _L)

    def src(j):
        return xyz_hbm.at[0, b, :, pl.ds(j * _C, _C)]

    cps = [pltpu.async_copy(src(j), bufs[j], sems[j]) for j in range(_NBUF)]

    accs = tuple(
        (
            jnp.full((_L,), _NEG_INF, jnp.float32),
            jnp.zeros((_L,), jnp.int32),
            jnp.zeros((_L,), jnp.float32),
            jnp.zeros((_L,), jnp.float32),
        )
        for _ in range(_U)
    )
    for j in range(_NBUF):
        cps[j].wait()
        accs = _scan_chunk_y(bufs[j], j * _C, accs, lanes)
    cp7 = pltpu.async_copy(src(_NCH - 1), bufs[6], sems[6])
    cp7.wait()
    accs = _scan_chunk_y(bufs[6], (_NCH - 1) * _C, accs, lanes)

    cy, idxv, cx, cz = _butterfly(_tree_combine(accs), lanes)

    accs2 = tuple(
        (
            jnp.full((_L,), _NEG_INF, jnp.float32),
            jnp.zeros((_L,), jnp.int32),
        )
        for _ in range(_U)
    )
    accs2 = _scan_chunk_d(bufs[6], (_NCH - 1) * _C, accs2, lanes, cx, cy, cz)
    cp6 = pltpu.async_copy(src(6), bufs[6], sems[6])
    for j in range(6):
        accs2 = _scan_chunk_d(bufs[j], j * _C, accs2, lanes, cx, cy, cz)
    cp6.wait()
    accs2 = _scan_chunk_d(bufs[6], 6 * _C, accs2, lanes, cx, cy, cz)

    _, idxv2 = _butterfly(_tree_combine(accs2), lanes)

    stage_v[...] = jnp.where(lanes == 0, idxv, idxv2)
    pltpu.sync_copy(stage_v, out_hbm.at[b])


@jax.jit
def kernel(xyz):
    mesh = plsc.VectorSubcoreMesh(core_axis_name="c", subcore_axis_name="s")
    fps = pl.kernel(
        _fps_body,
        out_type=jax.ShapeDtypeStruct((_B, _L), jnp.int32),
        mesh=mesh,
        compiler_params=pltpu.CompilerParams(
            use_tc_tiling_on_sc=True, needs_layout_passes=False
        ),
        scratch_types=(
            [pltpu.VMEM((3, _C), jnp.float32) for _ in range(_NBUF)]
            + [pltpu.VMEM((_L,), jnp.int32)]
            + [pltpu.SemaphoreType.DMA for _ in range(_NBUF)]
        ),
    )
    out = fps(xyz)
    return out[:, :2]


# speedup vs baseline: 1.4270x; 1.0011x over previous
import jax
import jax.numpy as jnp
from jax import lax
from jax.experimental import pallas as pl
from jax.experimental.pallas import tpu as pltpu
from jax.experimental.pallas import tpu_sc as plsc

_B = 32
_N = 16384
_L = 16
_U = 8
_C = 2048
_NCH = _N // _C
_NBUF = 7

_NEG_INF = float("-inf")


def _combine(a, b):
    take = (b[0] > a[0]) | ((b[0] == a[0]) & (b[1] < a[1]))
    return tuple(jnp.where(take, y, x) for x, y in zip(a, b))


def _butterfly(acc, lanes):
    for s in (8, 4, 2, 1):
        perm = lanes ^ s
        other = tuple(v.at[perm].get(mode="promise_in_bounds") for v in acc)
        acc = _combine(acc, other)
    return acc


def _tree_combine(accs):
    accs = list(accs)
    while len(accs) > 1:
        accs = [_combine(accs[k], accs[k + 1]) for k in range(0, len(accs), 2)]
    return accs[0]


def _scan_chunk_y(buf, base, accs, lanes):
    @plsc.parallel_loop(0, _C, step=_U * _L, carry=accs)
    def loop(i, accs):
        out = []
        for k in range(_U):
            sl = pl.ds(i + k * _L, _L)
            cand = (buf[1, sl], lanes + (base + i + k * _L), buf[0, sl], buf[2, sl])
            out.append(_combine(accs[k], cand))
        return tuple(out)

    return loop


def _scan_chunk_d(buf, base, accs, lanes, cx, cy, cz):
    @plsc.parallel_loop(0, _C, step=_U * _L, carry=accs)
    def loop(i, accs):
        out = []
        for k in range(_U):
            sl = pl.ds(i + k * _L, _L)
            dx = buf[0, sl] - cx
            dy = buf[1, sl] - cy
            dz = buf[2, sl] - cz
            d = dx * dx + dy * dy + dz * dz
            d = jnp.minimum(d, jnp.float32(1e10))
            out.append(_combine(accs[k], (d, lanes + (base + i + k * _L))))
        return tuple(out)

    return loop


def _fps_body(xyz_hbm, out_hbm, b0, b1, b2, b3, b4, b5, b6, stage_v,
              s0, s1, s2, s3, s4, s5, s6):
    c = lax.axis_index("c")
    s = lax.axis_index("s")
    b = s * 2 + c

    bufs = (b0, b1, b2, b3, b4, b5, b6)
    sems = (s0, s1, s2, s3, s4, s5, s6)
    lanes = lax.iota(jnp.int32, _L)

    def src(j):
        return xyz_hbm.at[0, b, :, pl.ds(j * _C, _C)]

    cps = [pltpu.async_copy(src(j), bufs[j], sems[j]) for j in range(_NBUF)]

    accs = tuple(
        (
            jnp.full((_L,), _NEG_INF, jnp.float32),
            jnp.zeros((_L,), jnp.int32),
            jnp.zeros((_L,), jnp.float32),
            jnp.zeros((_L,), jnp.float32),
        )
        for _ in range(_U)
    )
    for j in range(_NBUF):
        cps[j].wait()
        accs = _scan_chunk_y(bufs[j], j * _C, accs, lanes)
    cp7 = pltpu.async_copy(src(_NCH - 1), bufs[6], sems[6])
    cp7.wait()
    accs = _scan_chunk_y(bufs[6], (_NCH - 1) * _C, accs, lanes)

    cy, idxv, cx, cz = _butterfly(_tree_combine(accs), lanes)

    accs2 = tuple(
        (
            jnp.full((_L,), _NEG_INF, jnp.float32),
            jnp.zeros((_L,), jnp.int32),
        )
        for _ in range(_U)
    )
    accs2 = _scan_chunk_d(bufs[6], (_NCH - 1) * _C, accs2, lanes, cx, cy, cz)
    cp6 = pltpu.async_copy(src(6), bufs[6], sems[6])
    for j in range(6):
        accs2 = _scan_chunk_d(bufs[j], j * _C, accs2, lanes, cx, cy, cz)
    cp6.wait()
    accs2 = _scan_chunk_d(bufs[6], 6 * _C, accs2, lanes, cx, cy, cz)

    _, idxv2 = _butterfly(_tree_combine(accs2), lanes)

    stage_v[...] = jnp.where(lanes == 0, idxv, idxv2)
    pltpu.sync_copy(stage_v, out_hbm.at[b])


@jax.jit
def kernel(xyz):
    mesh = plsc.VectorSubcoreMesh(core_axis_name="c", subcore_axis_name="s")
    fps = pl.kernel(
        _fps_body,
        out_type=jax.ShapeDtypeStruct((_B, _L), jnp.int32),
        mesh=mesh,
        compiler_params=pltpu.CompilerParams(
            use_tc_tiling_on_sc=True,
            needs_layout_passes=False,
            skip_device_barrier=True,
            disable_bounds_checks=True,
            disable_semaphore_checks=True
        ),
        scratch_types=(
            [pltpu.VMEM((3, _C), jnp.float32) for _ in range(_NBUF)]
            + [pltpu.VMEM((_L,), jnp.int32)]
            + [pltpu.SemaphoreType.DMA for _ in range(_NBUF)]
        ),
    )
    out = fps(xyz)
    return out[:, :2]
